# baseline (speedup 1.0000x reference)
import functools

import jax
import jax.numpy as jnp
from jax import lax
from jax.experimental import pallas as pl
from jax.experimental.pallas import tpu as pltpu

_NUM_CLASSES = 21
_THRESHOLD = 0.5
_NEG_POS_RATIO = 3
_VAR0 = 0.1
_VAR1 = 0.2
_NCHUNK = 1
_SUB = 8


def _body(cw_ref, tg_ref, loc_ref, conf_ref, mn_ref,
          out_loc_ref, out_conf_ref, out_n_ref,
          bto_s, bti_s, er_s, pmx_s, pix_s, bpi_s, accf, acci,
          *, B, P, M, LAN):
    b = pl.program_id(0)
    c2 = pl.program_id(1)
    CHUNK = _SUB * LAN

    @pl.when(jnp.logical_and(b == 0, c2 == 0))
    def _init():
        accf[0] = 0.0
        accf[1] = 0.0
        acci[1] = 0

    def prior_idx3(c):
        s_io = lax.broadcasted_iota(jnp.int32, (M, _SUB, LAN), 1)
        l_io = lax.broadcasted_iota(jnp.int32, (M, _SUB, LAN), 2)
        return c * CHUNK + s_io * LAN + l_io

    @pl.when(c2 < _NCHUNK)
    def _phase1():
        c = c2
        cen = cw_ref[0, pl.ds(c * _SUB, _SUB), :]
        wid = cw_ref[1, pl.ds(c * _SUB, _SUB), :]
        pf_lo = cen - wid * 0.5
        pf_hi = cen + wid * 0.5
        ts = tg_ref[0, 0, :]
        te = tg_ref[0, 1, :]
        ts3 = ts[:, None, None]
        te3 = te[:, None, None]
        lo = jnp.maximum(ts3, pf_lo[None])
        hi = jnp.minimum(te3, pf_hi[None])
        inter = jnp.maximum(hi - lo, 0.0)
        union = (te3 - ts3) + (pf_hi - pf_lo)[None] - inter
        ov = inter / union

        miota = lax.broadcasted_iota(jnp.int32, (M, _SUB, LAN), 0)

        bto_c = jnp.max(ov, axis=0)
        bti_c = jnp.min(jnp.where(ov == bto_c[None], miota, M), axis=0)
        bto_s[pl.ds(c * _SUB, _SUB), :] = bto_c
        bti_s[pl.ds(c * _SUB, _SUB), :] = bti_c

        cm = jnp.max(ov, axis=(1, 2))
        pidx = prior_idx3(c)
        cil = jnp.min(jnp.where(ov == cm[:, None, None], pidx, P),
                      axis=(1, 2))
        pmx_s[pl.ds(c, 1), :] = cm.reshape(1, M)
        pix_s[pl.ds(c, 1), :] = cil.reshape(1, M)

        @pl.when(c == _NCHUNK - 1)
        def _merge():
            vals = pmx_s[:, :]
            idxs = pix_s[:, :]
            gmax = jnp.max(vals, axis=0)
            ciota = lax.broadcasted_iota(jnp.int32, (_NCHUNK, M), 0)
            cfirst = jnp.min(
                jnp.where(vals == gmax[None, :], ciota, _NCHUNK), axis=0)
            bpi = jnp.zeros((M,), jnp.int32)
            for cc in range(_NCHUNK):
                bpi = jnp.where(cfirst == cc, idxs[cc, :], bpi)
            bpi_s[0, :] = bpi

    @pl.when(c2 >= _NCHUNK)
    def _phase2():
        c = c2 - _NCHUNK

        @pl.when(c == 0)
        def _reset_row():
            accf[2] = 0.0
            acci[0] = 0

        cen = cw_ref[0, pl.ds(c * _SUB, _SUB), :]
        wid = cw_ref[1, pl.ds(c * _SUB, _SUB), :]
        ts = tg_ref[0, 0, :]
        te = tg_ref[0, 1, :]
        lab = tg_ref[0, 2, :]

        bto_c = bto_s[pl.ds(c * _SUB, _SUB), :]
        bti_c = bti_s[pl.ds(c * _SUB, _SUB), :]

        bpi = bpi_s[0, :]
        pidx = prior_idx3(c)
        mhit = bpi[:, None, None] == pidx
        miota = lax.broadcasted_iota(jnp.int32, (M, _SUB, LAN), 0)
        m_last = jnp.max(jnp.where(mhit, miota, -1), axis=0)
        hit = m_last >= 0
        bto_c = jnp.where(hit, 2.0, bto_c)
        bti_c = jnp.where(hit, m_last, bti_c)

        msel = bti_c[None] == miota
        ms = jnp.sum(jnp.where(msel, ts[:, None, None], 0.0), axis=0)
        me = jnp.sum(jnp.where(msel, te[:, None, None], 0.0), axis=0)
        ml = jnp.sum(jnp.where(msel, lab[:, None, None], 0.0), axis=0)

        conf_c = jnp.where(bto_c < _THRESHOLD, 0.0, ml + 1.0)
        pos_c = conf_c > 0.0
        acci[0] += jnp.sum(pos_c.astype(jnp.int32))

        g_c = ((ms + me) * 0.5 - cen) / (_VAR0 * wid)
        g_w = jnp.log((me - ms) / wid) / _VAR1
        d0 = loc_ref[0, 0] - g_c
        d1 = loc_ref[0, 1] - g_w
        sl = (jnp.where(jnp.abs(d0) < 1.0, 0.5 * d0 * d0, jnp.abs(d0) - 0.5)
              + jnp.where(jnp.abs(d1) < 1.0, 0.5 * d1 * d1, jnp.abs(d1) - 0.5))
        accf[0] += jnp.sum(jnp.where(pos_c, sl, 0.0))

        cd = conf_ref[0]
        xmax = jnp.max(cd, axis=0)
        se = jnp.sum(jnp.exp(cd - xmax[None]), axis=0)
        liota = lax.broadcasted_iota(jnp.int32, (_NUM_CLASSES, _SUB, LAN), 0)
        ci = conf_c.astype(jnp.int32)
        gathered = jnp.sum(jnp.where(liota == ci[None], cd, 0.0), axis=0)
        e_c = jnp.log(se) + xmax - gathered
        accf[2] += jnp.sum(jnp.where(pos_c, e_c, 0.0))
        er_s[pl.ds(c * _SUB, _SUB), :] = jnp.where(pos_c, 0.0, e_c)

        @pl.when(c == _NCHUNK - 1)
        def _select():
            np_b = acci[0]
            k = jnp.clip(np_b * _NEG_POS_RATIO, mn_ref[0], P - 1)
            ev = er_s[:, :]
            bits = lax.bitcast_convert_type(ev, jnp.int32)

            def rs_body(i, T):
                cand = T | (jnp.int32(1) << (30 - i))
                cnt = jnp.sum((bits >= cand).astype(jnp.int32))
                return jnp.where(cnt >= k, cand, T)

            T = lax.fori_loop(0, 31, rs_body, jnp.int32(0))
            v = jnp.max(jnp.where(bits == T, ev, -jnp.inf))
            gt = bits > T
            cnt_gt = jnp.sum(gt.astype(jnp.int32))
            sum_gt = jnp.sum(jnp.where(gt, ev, 0.0))
            neg_sum = sum_gt + (k - cnt_gt).astype(jnp.float32) * v
            accf[1] += accf[2] + neg_sum
            acci[1] += np_b

            @pl.when(b == B - 1)
            def _finish():
                out_loc_ref[:, :] = jnp.full((1, 1), accf[0], jnp.float32)
                out_conf_ref[:, :] = jnp.full((1, 1), accf[1], jnp.float32)
                out_n_ref[:, :] = jnp.full((1, 1), acci[1], jnp.int32)


@functools.partial(jax.jit, static_argnames=())
def kernel(loc_data, conf_data, priors, targets, min_neg):
    B, P, _ = loc_data.shape
    M = targets.shape[1]
    C = conf_data.shape[2]
    LAN = P // (_NCHUNK * _SUB)
    ROWS = P // LAN
    cw = priors.T.reshape(2, ROWS, LAN)
    mn = jnp.reshape(jnp.asarray(min_neg, jnp.int32), (1,))

    def run_part(loc_h, conf_h, tg_h):
        Bh = loc_h.shape[0]
        loc_t = jnp.transpose(loc_h, (0, 2, 1)).reshape(Bh, 2, ROWS, LAN)
        conf_t = jnp.transpose(conf_h, (0, 2, 1)).reshape(Bh, C, ROWS, LAN)
        tg = jnp.transpose(tg_h, (0, 2, 1))
        grid = (Bh, 2 * _NCHUNK)
        return pl.pallas_call(
            functools.partial(_body, B=Bh, P=P, M=M, LAN=LAN),
            grid=grid,
            in_specs=[
                pl.BlockSpec((2, ROWS, LAN), lambda b, c: (0, 0, 0)),
                pl.BlockSpec((1, 3, M), lambda b, c: (b, 0, 0)),
                pl.BlockSpec((1, 2, _SUB, LAN),
                             lambda b, c: (b, 0, jnp.maximum(c - _NCHUNK, 0), 0)),
                pl.BlockSpec((1, _NUM_CLASSES, _SUB, LAN),
                             lambda b, c: (b, 0, jnp.maximum(c - _NCHUNK, 0), 0)),
                pl.BlockSpec(memory_space=pltpu.SMEM),
            ],
            out_specs=[
                pl.BlockSpec((1, 1), lambda b, c: (0, 0)),
                pl.BlockSpec((1, 1), lambda b, c: (0, 0)),
                pl.BlockSpec((1, 1), lambda b, c: (0, 0)),
            ],
            out_shape=[
                jax.ShapeDtypeStruct((1, 1), jnp.float32),
                jax.ShapeDtypeStruct((1, 1), jnp.float32),
                jax.ShapeDtypeStruct((1, 1), jnp.int32),
            ],
            scratch_shapes=[
                pltpu.VMEM((ROWS, LAN), jnp.float32),
                pltpu.VMEM((ROWS, LAN), jnp.int32),
                pltpu.VMEM((ROWS, LAN), jnp.float32),
                pltpu.VMEM((_NCHUNK, M), jnp.float32),
                pltpu.VMEM((_NCHUNK, M), jnp.int32),
                pltpu.VMEM((1, M), jnp.int32),
                pltpu.SMEM((4,), jnp.float32),
                pltpu.SMEM((2,), jnp.int32),
            ],
        )(cw, tg, loc_t, conf_t, mn)

    H = B // 2
    l0, c0, n0 = run_part(loc_data[:H], conf_data[:H], targets[:H])
    l1, c1, n1 = run_part(loc_data[H:], conf_data[H:], targets[H:])
    n = (n0 + n1).astype(jnp.float32).reshape(())
    return (((l0 + l1).reshape(()) / n), ((c0 + c1).reshape(()) / n))


# speedup vs baseline: 1.1645x; 1.1645x over previous
import functools

import jax
import jax.numpy as jnp
from jax import lax
from jax.experimental import pallas as pl
from jax.experimental.pallas import tpu as pltpu

_NUM_CLASSES = 21
_THRESHOLD = 0.5
_NEG_POS_RATIO = 3
_VAR0 = 0.1
_VAR1 = 0.2
_NCHUNK = 1
_SUB = 8


def _body(cw_ref, tg_ref, loc_ref, conf_ref, mn_ref,
          out_loc_ref, out_conf_ref,
          bto_s, bti_s, er_s, pmx_s, pix_s, bpi_s, accf, acci,
          *, B, P, M, LAN):
    b = pl.program_id(0)
    c2 = pl.program_id(1)
    CHUNK = _SUB * LAN

    @pl.when(jnp.logical_and(b == 0, c2 == 0))
    def _init():
        accf[0] = 0.0
        accf[1] = 0.0
        acci[1] = 0

    def prior_idx3(c):
        s_io = lax.broadcasted_iota(jnp.int32, (M, _SUB, LAN), 1)
        l_io = lax.broadcasted_iota(jnp.int32, (M, _SUB, LAN), 2)
        return c * CHUNK + s_io * LAN + l_io

    @pl.when(c2 < _NCHUNK)
    def _phase1():
        c = c2
        cen = cw_ref[0, pl.ds(c * _SUB, _SUB), :]
        wid = cw_ref[1, pl.ds(c * _SUB, _SUB), :]
        pf_lo = cen - wid * 0.5
        pf_hi = cen + wid * 0.5
        ts = tg_ref[0, 0, :]
        te = tg_ref[0, 1, :]
        ts3 = ts[:, None, None]
        te3 = te[:, None, None]
        lo = jnp.maximum(ts3, pf_lo[None])
        hi = jnp.minimum(te3, pf_hi[None])
        inter = jnp.maximum(hi - lo, 0.0)
        union = (te3 - ts3) + (pf_hi - pf_lo)[None] - inter
        ov = inter / union

        miota = lax.broadcasted_iota(jnp.int32, (M, _SUB, LAN), 0)

        bto_c = jnp.max(ov, axis=0)
        bti_c = jnp.min(jnp.where(ov == bto_c[None], miota, M), axis=0)
        bto_s[pl.ds(c * _SUB, _SUB), :] = bto_c
        bti_s[pl.ds(c * _SUB, _SUB), :] = bti_c

        cm = jnp.max(ov, axis=(1, 2))
        pidx = prior_idx3(c)
        cil = jnp.min(jnp.where(ov == cm[:, None, None], pidx, P),
                      axis=(1, 2))
        pmx_s[pl.ds(c, 1), :] = cm.reshape(1, M)
        pix_s[pl.ds(c, 1), :] = cil.reshape(1, M)

        @pl.when(c == _NCHUNK - 1)
        def _merge():
            vals = pmx_s[:, :]
            idxs = pix_s[:, :]
            gmax = jnp.max(vals, axis=0)
            ciota = lax.broadcasted_iota(jnp.int32, (_NCHUNK, M), 0)
            cfirst = jnp.min(
                jnp.where(vals == gmax[None, :], ciota, _NCHUNK), axis=0)
            bpi = jnp.zeros((M,), jnp.int32)
            for cc in range(_NCHUNK):
                bpi = jnp.where(cfirst == cc, idxs[cc, :], bpi)
            bpi_s[0, :] = bpi

    @pl.when(c2 >= _NCHUNK)
    def _phase2():
        c = c2 - _NCHUNK

        @pl.when(c == 0)
        def _reset_row():
            accf[2] = 0.0
            acci[0] = 0

        cen = cw_ref[0, pl.ds(c * _SUB, _SUB), :]
        wid = cw_ref[1, pl.ds(c * _SUB, _SUB), :]
        ts = tg_ref[0, 0, :]
        te = tg_ref[0, 1, :]
        lab = tg_ref[0, 2, :]

        bto_c = bto_s[pl.ds(c * _SUB, _SUB), :]
        bti_c = bti_s[pl.ds(c * _SUB, _SUB), :]

        bpi = bpi_s[0, :]
        pidx = prior_idx3(c)
        mhit = bpi[:, None, None] == pidx
        miota = lax.broadcasted_iota(jnp.int32, (M, _SUB, LAN), 0)
        m_last = jnp.max(jnp.where(mhit, miota, -1), axis=0)
        hit = m_last >= 0
        bto_c = jnp.where(hit, 2.0, bto_c)
        bti_c = jnp.where(hit, m_last, bti_c)

        msel = bti_c[None] == miota
        ms = jnp.sum(jnp.where(msel, ts[:, None, None], 0.0), axis=0)
        me = jnp.sum(jnp.where(msel, te[:, None, None], 0.0), axis=0)
        ml = jnp.sum(jnp.where(msel, lab[:, None, None], 0.0), axis=0)

        conf_c = jnp.where(bto_c < _THRESHOLD, 0.0, ml + 1.0)
        pos_c = conf_c > 0.0
        acci[0] += jnp.sum(pos_c.astype(jnp.int32))

        g_c = ((ms + me) * 0.5 - cen) / (_VAR0 * wid)
        g_w = jnp.log((me - ms) / wid) / _VAR1
        d0 = loc_ref[0, 0] - g_c
        d1 = loc_ref[0, 1] - g_w
        sl = (jnp.where(jnp.abs(d0) < 1.0, 0.5 * d0 * d0, jnp.abs(d0) - 0.5)
              + jnp.where(jnp.abs(d1) < 1.0, 0.5 * d1 * d1, jnp.abs(d1) - 0.5))
        accf[0] += jnp.sum(jnp.where(pos_c, sl, 0.0))

        cd = conf_ref[0]
        xmax = jnp.max(cd, axis=0)
        se = jnp.sum(jnp.exp(cd - xmax[None]), axis=0)
        liota = lax.broadcasted_iota(jnp.int32, (_NUM_CLASSES, _SUB, LAN), 0)
        ci = conf_c.astype(jnp.int32)
        gathered = jnp.sum(jnp.where(liota == ci[None], cd, 0.0), axis=0)
        e_c = jnp.log(se) + xmax - gathered
        accf[2] += jnp.sum(jnp.where(pos_c, e_c, 0.0))
        er_s[pl.ds(c * _SUB, _SUB), :] = jnp.where(pos_c, 0.0, e_c)

        @pl.when(c == _NCHUNK - 1)
        def _select():
            np_b = acci[0]
            k = jnp.clip(np_b * _NEG_POS_RATIO, mn_ref[0], P - 1)
            ev = er_s[:, :]
            bits = lax.bitcast_convert_type(ev, jnp.int32)

            def rs_body(i, T):
                cand = T | (jnp.int32(1) << (30 - i))
                cnt = jnp.sum((bits >= cand).astype(jnp.int32))
                return jnp.where(cnt >= k, cand, T)

            T = lax.fori_loop(0, 31, rs_body, jnp.int32(0))
            v = jnp.max(jnp.where(bits == T, ev, -jnp.inf))
            gt = bits > T
            cnt_gt = jnp.sum(gt.astype(jnp.int32))
            sum_gt = jnp.sum(jnp.where(gt, ev, 0.0))
            neg_sum = sum_gt + (k - cnt_gt).astype(jnp.float32) * v
            accf[1] += accf[2] + neg_sum
            acci[1] += np_b

            @pl.when(b == B - 1)
            def _finish():
                n = acci[1].astype(jnp.float32)
                out_loc_ref[:, :] = jnp.full((1, 1), accf[0] / n, jnp.float32)
                out_conf_ref[:, :] = jnp.full((1, 1), accf[1] / n, jnp.float32)


@functools.partial(jax.jit, static_argnames=())
def kernel(loc_data, conf_data, priors, targets, min_neg):
    B, P, _ = loc_data.shape
    M = targets.shape[1]
    C = conf_data.shape[2]
    LAN = P // (_NCHUNK * _SUB)
    ROWS = P // LAN
    cw = priors.T.reshape(2, ROWS, LAN)
    tg = jnp.transpose(targets, (0, 2, 1))
    loc_t = jnp.transpose(loc_data, (0, 2, 1)).reshape(B, 2, ROWS, LAN)
    conf_t = jnp.transpose(conf_data, (0, 2, 1)).reshape(B, C, ROWS, LAN)
    mn = jnp.reshape(jnp.asarray(min_neg, jnp.int32), (1,))

    grid = (B, 2 * _NCHUNK)
    out_loc, out_conf = pl.pallas_call(
        functools.partial(_body, B=B, P=P, M=M, LAN=LAN),
        grid=grid,
        in_specs=[
            pl.BlockSpec((2, ROWS, LAN), lambda b, c: (0, 0, 0)),
            pl.BlockSpec((1, 3, M), lambda b, c: (b, 0, 0)),
            pl.BlockSpec((1, 2, _SUB, LAN),
                         lambda b, c: (b, 0, jnp.maximum(c - _NCHUNK, 0), 0)),
            pl.BlockSpec((1, _NUM_CLASSES, _SUB, LAN),
                         lambda b, c: (b, 0, jnp.maximum(c - _NCHUNK, 0), 0)),
            pl.BlockSpec(memory_space=pltpu.SMEM),
        ],
        out_specs=[
            pl.BlockSpec((1, 1), lambda b, c: (0, 0)),
            pl.BlockSpec((1, 1), lambda b, c: (0, 0)),
        ],
        out_shape=[
            jax.ShapeDtypeStruct((1, 1), jnp.float32),
            jax.ShapeDtypeStruct((1, 1), jnp.float32),
        ],
        scratch_shapes=[
            pltpu.VMEM((ROWS, LAN), jnp.float32),
            pltpu.VMEM((ROWS, LAN), jnp.int32),
            pltpu.VMEM((ROWS, LAN), jnp.float32),
            pltpu.VMEM((_NCHUNK, M), jnp.float32),
            pltpu.VMEM((_NCHUNK, M), jnp.int32),
            pltpu.VMEM((1, M), jnp.int32),
            pltpu.SMEM((4,), jnp.float32),
            pltpu.SMEM((2,), jnp.int32),
        ],
    )(cw, tg, loc_t, conf_t, mn)
    return (out_loc.reshape(()), out_conf.reshape(()))
